# baseline (speedup 1.0000x reference)
import functools

import jax
import jax.numpy as jnp
from jax import lax
from jax.experimental import pallas as pl
from jax.experimental.pallas import tpu as pltpu
from jax.experimental.pallas import tpu_sc as plsc

B = 4096
L = 50
D = 64
NC = 2
NS = 16
NW = NC * NS
BPW = B // NW
G = 16
CH = BPW // G
ROWS = G * L
NV = D // 16

_mesh = plsc.VectorSubcoreMesh(core_axis_name="c", subcore_axis_name="s")


@functools.partial(
    pl.kernel,
    mesh=_mesh,
    out_type=jax.ShapeDtypeStruct((B * L, D), jnp.float32),
    scratch_types=[
        pltpu.VMEM((ROWS,), jnp.int32),
        pltpu.VMEM((G,), jnp.int32),
        pltpu.VMEM((ROWS, D), jnp.float32),
        pltpu.VMEM((G, D), jnp.float32),
        pltpu.VMEM((100, D), jnp.float32),
        pltpu.SemaphoreType.DMA,
        pltpu.SemaphoreType.DMA,
    ],
    compiler_params=pltpu.CompilerParams(use_tc_tiling_on_sc=False),
)
def _hybrid_emb(tokens_hbm, labels_hbm, tok_tab, sty_tab, pos_tab, out_hbm,
                tok_idx, lab_idx, buf, sty, pos, sem_tok, sem_sty):
    wid = lax.axis_index("s") * NC + lax.axis_index("c")

    pltpu.sync_copy(pos_tab, pos)

    def chunk_body(c, carry):
        base_b = wid * BPW + c * G
        base_r = base_b * L
        pltpu.sync_copy(tokens_hbm.at[pl.ds(base_r, ROWS)], tok_idx)
        pltpu.sync_copy(labels_hbm.at[pl.ds(base_b, G)], lab_idx)
        cp_tok = pltpu.async_copy(tok_tab.at[tok_idx], buf, sem_tok)
        cp_sty = pltpu.async_copy(sty_tab.at[lab_idx], sty, sem_sty)
        cp_tok.wait()
        cp_sty.wait()

        def g_body(g, carry_g):
            def l_body(l, carry_l):
                r = g * L + l
                for j in range(NV):
                    sl = pl.ds(j * 16, 16)
                    buf[r, sl] = buf[r, sl] + pos[l, sl] + sty[g, sl]
                return carry_l
            return lax.fori_loop(0, L, l_body, carry_g)

        lax.fori_loop(0, G, g_body, 0)
        pltpu.sync_copy(buf, out_hbm.at[pl.ds(base_r, ROWS)])
        return carry

    lax.fori_loop(0, CH, chunk_body, 0)


def kernel(tokens, labels, token_table, style_table, posit_table):
    out = _hybrid_emb(
        tokens.reshape(-1).astype(jnp.int32),
        labels.astype(jnp.int32),
        token_table,
        style_table,
        posit_table,
    )
    return out.reshape(B, L, D)


# speedup vs baseline: 1.4502x; 1.4502x over previous
import functools

import jax
import jax.numpy as jnp
from jax import lax
from jax.experimental import pallas as pl
from jax.experimental.pallas import tpu as pltpu
from jax.experimental.pallas import tpu_sc as plsc

B = 4096
L = 50
D = 64
V = 1000000
VB = 8192
NC = 2
NS = 16
NW = NC * NS
BPW = B // NW
G = 8
CH = BPW // G
ROWS = G * L
NV = D // 16

def _transpose_body(t_ref, o_ref):
    o_ref[:, 0:D] = t_ref[...].T


_tc_transpose = pl.pallas_call(
    _transpose_body,
    grid=((V + VB - 1) // VB,),
    in_specs=[pl.BlockSpec((D, VB), lambda i: (0, i))],
    out_specs=pl.BlockSpec((VB, 2 * D), lambda i: (i, 0)),
    out_shape=jax.ShapeDtypeStruct((V, 2 * D), jnp.float32),
)

_mesh = plsc.VectorSubcoreMesh(core_axis_name="c", subcore_axis_name="s")


@functools.partial(
    pl.kernel,
    mesh=_mesh,
    out_type=jax.ShapeDtypeStruct((B * L, D), jnp.float32),
    scratch_types=[
        pltpu.VMEM((ROWS,), jnp.int32),
        pltpu.VMEM((G,), jnp.int32),
        pltpu.VMEM((ROWS, 2 * D), jnp.float32),
        pltpu.VMEM((ROWS, D), jnp.float32),
        pltpu.VMEM((G, D), jnp.float32),
        pltpu.VMEM((100, D), jnp.float32),
        pltpu.SemaphoreType.DMA,
        pltpu.SemaphoreType.DMA,
    ],
    compiler_params=pltpu.CompilerParams(use_tc_tiling_on_sc=False),
)
def _hybrid_emb(tokens_hbm, labels_hbm, tok_tab, sty_tab, pos_tab, out_hbm,
                tok_idx, lab_idx, buf, outb, sty, pos, sem_tok, sem_sty):
    wid = lax.axis_index("s") * NC + lax.axis_index("c")

    pltpu.sync_copy(pos_tab, pos)

    def chunk_body(c, carry):
        base_b = wid * BPW + c * G
        base_r = base_b * L
        pltpu.sync_copy(tokens_hbm.at[pl.ds(base_r, ROWS)], tok_idx)
        pltpu.sync_copy(labels_hbm.at[pl.ds(base_b, G)], lab_idx)
        cp_tok = pltpu.async_copy(tok_tab.at[tok_idx], buf, sem_tok)
        cp_sty = pltpu.async_copy(sty_tab.at[lab_idx], sty, sem_sty)
        cp_tok.wait()
        cp_sty.wait()

        def g_body(g, carry_g):
            def l_body(l, carry_l):
                r = g * L + l
                for j in range(NV):
                    sl = pl.ds(j * 16, 16)
                    outb[r, sl] = buf[r, sl] + pos[l, sl] + sty[g, sl]
                return carry_l
            return lax.fori_loop(0, L, l_body, carry_g)

        lax.fori_loop(0, G, g_body, 0)
        pltpu.sync_copy(outb, out_hbm.at[pl.ds(base_r, ROWS)])
        return carry

    lax.fori_loop(0, CH, chunk_body, 0)


def kernel(tokens, labels, token_table, style_table, posit_table):
    table_pad = _tc_transpose(token_table.T)
    out = _hybrid_emb(
        tokens.reshape(-1).astype(jnp.int32),
        labels.astype(jnp.int32),
        table_pad,
        style_table,
        posit_table,
    )
    return out.reshape(B, L, D)
